# Initial kernel scaffold; baseline (speedup 1.0000x reference)
import jax
import jax.numpy as jnp
from jax.experimental import pallas as pl


def kernel(x, edge_attr, edge_index, batch, W_emb, b_emb, W_r1, b_r1, W_r2, b_r2, Wq, bq, Wk, bk, Wv, bv, We, Wc, bc, Wm, bm, ln_g, ln_b, Wmsg, bmsg, mln_g, mln_b, bn_g, bn_b, W_fc, b_fc, W_out, b_out):
    raise NotImplementedError("write your pallas kernel here")


# speedup vs baseline: 3.3138x; 3.3138x over previous
import functools
import math

import jax
import jax.numpy as jnp
from jax import lax
from jax.experimental import pallas as pl
from jax.experimental.pallas import tpu as pltpu
from jax.experimental.pallas import tpu_sc as plsc

N = 10000
E = 160000
NF = 128
C = 128
L = 5
G = 64

NC, NS = 2, 16
NW = NC * NS
K = 128
EP = 163840
CHUNKS = EP // (NW * K)
NPAD = 10240
ROWS_PER_TILE = NPAD // NS
BE = 2048


def _ln_rows(x, g, b):
    mu = jnp.mean(x, axis=1, keepdims=True)
    var = jnp.mean((x - mu) * (x - mu), axis=1, keepdims=True)
    return (x - mu) * lax.rsqrt(var + 1e-5) * g + b


def _embed_body(x_ref, w_ref, b_ref, o_ref):
    o_ref[...] = (
        jnp.dot(x_ref[...], w_ref[...], preferred_element_type=jnp.float32)
        + b_ref[...])


def _ef_body(ea_ref, w1_ref, b1_ref, w2_ref, b2_ref, o_ref):
    ea = ea_ref[...]
    d = jnp.sqrt(jnp.sum(ea * ea, axis=1, keepdims=True))
    step = 8.0 / 127.0
    centers = lax.broadcasted_iota(jnp.int32, (1, 128), 1).astype(jnp.float32) * step
    gamma = 1.0 / (step * step)
    diff = d - centers
    rbf = jnp.exp(-gamma * diff * diff)
    h = jnp.dot(rbf, w1_ref[...], preferred_element_type=jnp.float32) + b1_ref[...]
    h = jnp.maximum(h, 0.0) + jnp.log(1.0 + jnp.exp(-jnp.abs(h)))
    o_ref[...] = (
        jnp.dot(h, w2_ref[...], preferred_element_type=jnp.float32) + b2_ref[...])


def _node_body(nf_ref, wq_ref, bq_ref, wk_ref, bk_ref, wv_ref, bv_ref,
               tdst_ref, tsrc_ref):
    nf = nf_ref[...]
    q = jnp.dot(nf, wq_ref[...], preferred_element_type=jnp.float32) + bq_ref[...]
    k = jnp.dot(nf, wk_ref[...], preferred_element_type=jnp.float32) + bk_ref[...]
    v = jnp.dot(nf, wv_ref[...], preferred_element_type=jnp.float32) + bv_ref[...]
    tdst_ref[:, 0:C] = q
    tdst_ref[:, C:2 * C] = k
    tdst_ref[:, 2 * C:3 * C] = v
    tsrc_ref[:, 0:C] = k
    tsrc_ref[:, C:2 * C] = v


def _edge_body(gd_ref, gs_ref, ef_ref, we_ref, wm_ref, bm_ref, wmsg_ref,
               bmsg_ref, lng_ref, lnb_ref, mlng_ref, mlnb_ref, msg_ref):
    ef = ef_ref[...]
    e = jnp.dot(ef, we_ref[...], preferred_element_type=jnp.float32)
    gd = gd_ref[...]
    gs = gs_ref[...]
    qd = gd[:, 0:C]
    kd = gd[:, C:2 * C]
    vd = gd[:, 2 * C:3 * C]
    ks = gs[:, 0:C]
    vs = gs[:, C:2 * C]
    scale = 1.0 / math.sqrt(3.0 * C)
    alpha = jnp.concatenate([qd * kd, qd * ks, qd * e], axis=1) * scale
    gate = jax.nn.sigmoid(_ln_rows(alpha, lng_ref[...], lnb_ref[...]))
    hin = jnp.concatenate([vd, vs, e], axis=1)
    h = jnp.dot(hin, wm_ref[...], preferred_element_type=jnp.float32) + bm_ref[...]
    m = jnp.dot(h * gate, wmsg_ref[...],
                preferred_element_type=jnp.float32) + bmsg_ref[...]
    msg_ref[...] = _ln_rows(m, mlng_ref[...], mlnb_ref[...])


def _update_body(p_ref, wc_ref, bc_ref, bng_ref, bnb_ref, nf_ref):
    agg = p_ref[0:N, :]
    out = jnp.dot(agg, wc_ref[...], preferred_element_type=jnp.float32) + bc_ref[...]
    mu = jnp.mean(out, axis=0, keepdims=True)
    var = jnp.mean((out - mu) * (out - mu), axis=0, keepdims=True)
    o = (out - mu) * lax.rsqrt(var + 1e-5) * bng_ref[...] + bnb_ref[...]
    nf_ref[...] = o * jax.nn.sigmoid(o)


def _pool_body(nf_ref, b_ref, wfc_ref, bfc_ref, wout_ref, bout_ref, o_ref):
    nf = nf_ref[...]
    b = b_ref[...]
    seg = lax.broadcasted_iota(jnp.int32, (1, 128), 1)
    oh = (b == seg).astype(jnp.float32)
    pooled = lax.dot_general(oh, nf, (((0,), (0,)), ((), ())),
                             preferred_element_type=jnp.float32)
    ones = jnp.ones((N, 1), jnp.float32)
    cnt = lax.dot_general(oh, ones, (((0,), (0,)), ((), ())),
                          preferred_element_type=jnp.float32)
    pm = pooled / jnp.maximum(cnt, 1.0)
    feat = jnp.dot(pm, wfc_ref[...], preferred_element_type=jnp.float32) + bfc_ref[...]
    feat = feat * jax.nn.sigmoid(feat)
    o_ref[...] = (
        jnp.dot(feat, wout_ref[...], preferred_element_type=jnp.float32)
        + bout_ref[...])


@functools.cache
def _sc_mesh():
    return plsc.VectorSubcoreMesh(
        core_axis_name="c", subcore_axis_name="s",
        num_cores=NC, num_subcores=NS)


@functools.cache
def _make_sc_gather():
    @functools.partial(
        pl.kernel,
        out_type=[jax.ShapeDtypeStruct((EP, 3 * C), jnp.float32),
                  jax.ShapeDtypeStruct((EP, 2 * C), jnp.float32)],
        mesh=_sc_mesh(),
        scratch_types=[
            pltpu.VMEM((K,), jnp.int32),
            pltpu.VMEM((K,), jnp.int32),
            pltpu.VMEM((K, 3 * C), jnp.float32),
            pltpu.VMEM((K, 2 * C), jnp.float32),
            pltpu.SemaphoreType.DMA,
            pltpu.SemaphoreType.DMA,
        ],
    )
    def _sc_gather(tdst_hbm, tsrc_hbm, dst_hbm, src_hbm, gdst_hbm, gsrc_hbm,
                   idxd, idxs, rowsd, rowss, semd, sems):
        wid = lax.axis_index("s") * NC + lax.axis_index("c")
        base = wid * (CHUNKS * K)

        def body(i, carry):
            off = pl.multiple_of(base + i * K, 8)
            pltpu.sync_copy(dst_hbm.at[pl.ds(off, K)], idxd)
            pltpu.sync_copy(src_hbm.at[pl.ds(off, K)], idxs)
            cd = pltpu.async_copy(tdst_hbm.at[idxd], rowsd, semd)
            cs = pltpu.async_copy(tsrc_hbm.at[idxs], rowss, sems)
            cd.wait()
            cs.wait()
            pltpu.sync_copy(rowsd, gdst_hbm.at[pl.ds(off, K)])
            pltpu.sync_copy(rowss, gsrc_hbm.at[pl.ds(off, K)])
            return carry

        lax.fori_loop(0, CHUNKS, body, 0)

    return _sc_gather


@functools.cache
def _make_sc_scatter():
    chunks = EP // (NS * K)

    @functools.partial(
        pl.kernel,
        out_type=jax.ShapeDtypeStruct((NPAD, C), jnp.float32),
        mesh=plsc.VectorSubcoreMesh(
            core_axis_name="c", subcore_axis_name="s",
            num_cores=1, num_subcores=NS),
        scratch_types=[
            pltpu.VMEM((K,), jnp.int32),
            pltpu.VMEM((K, C), jnp.float32),
            pltpu.VMEM((64, C), jnp.float32),
            pltpu.VMEM_SHARED((NPAD, C), jnp.float32),
        ],
    )
    def _sc_scatter(msg_hbm, dst_hbm, zeros_hbm, out_hbm, idxv, mv, tbuf, acc):
        s = lax.axis_index("s")
        row0 = pl.multiple_of(s * ROWS_PER_TILE, 8)
        pltpu.sync_copy(zeros_hbm, tbuf)

        def zbody(j, carry):
            r = pl.multiple_of(row0 + j * 64, 8)
            pltpu.sync_copy(tbuf, acc.at[pl.ds(r, 64)])
            return carry

        lax.fori_loop(0, ROWS_PER_TILE // 64, zbody, 0)
        plsc.subcore_barrier()

        base = s * (chunks * K)

        def body(i, carry):
            off = pl.multiple_of(base + i * K, 8)
            pltpu.sync_copy(dst_hbm.at[pl.ds(off, K)], idxv)
            pltpu.sync_copy(msg_hbm.at[pl.ds(off, K)], mv)
            pltpu.sync_copy(mv, acc.at[idxv], add=True)
            return carry

        lax.fori_loop(0, chunks, body, 0)
        plsc.subcore_barrier()

        def obody(j, carry):
            r = pl.multiple_of(row0 + j * 64, 8)
            pltpu.sync_copy(acc.at[pl.ds(r, 64)], tbuf)
            pltpu.sync_copy(tbuf, out_hbm.at[pl.ds(r, 64)])
            return carry

        lax.fori_loop(0, ROWS_PER_TILE // 64, obody, 0)

    return _sc_scatter


def _row(v):
    return v.reshape(1, -1)


def kernel(x, edge_attr, edge_index, batch, W_emb, b_emb, W_r1, b_r1, W_r2,
           b_r2, Wq, bq, Wk, bk, Wv, bv, We, Wc, bc, Wm, bm, ln_g, ln_b,
           Wmsg, bmsg, mln_g, mln_b, bn_g, bn_b, W_fc, b_fc, W_out, b_out):
    f32 = jnp.float32
    src = edge_index[0]
    dst = edge_index[1]
    pad = EP - E
    gfill = jnp.arange(pad, dtype=jnp.int32) % N
    sfill = N + jnp.arange(pad, dtype=jnp.int32) % (NPAD - N)
    src_g = jnp.concatenate([src, gfill])
    dst_g = jnp.concatenate([dst, gfill])
    dst_s = jnp.concatenate([dst, sfill])
    ea_p = jnp.concatenate([edge_attr, jnp.zeros((pad, 3), f32)])
    zeros_blk = jnp.zeros((64, C), f32)

    nf = pl.pallas_call(
        _embed_body,
        out_shape=jax.ShapeDtypeStruct((N, NF), f32),
    )(x, W_emb, _row(b_emb))

    ef = pl.pallas_call(
        _ef_body,
        grid=(EP // 4096,),
        in_specs=[
            pl.BlockSpec((4096, 3), lambda i: (i, 0)),
            pl.BlockSpec((NF, NF), lambda i: (0, 0)),
            pl.BlockSpec((1, NF), lambda i: (0, 0)),
            pl.BlockSpec((NF, NF), lambda i: (0, 0)),
            pl.BlockSpec((1, NF), lambda i: (0, 0)),
        ],
        out_specs=pl.BlockSpec((4096, NF), lambda i: (i, 0)),
        out_shape=jax.ShapeDtypeStruct((EP, NF), f32),
    )(ea_p, W_r1, _row(b_r1), W_r2, _row(b_r2))

    for l in range(L):
        tdst, tsrc = pl.pallas_call(
            _node_body,
            out_shape=[jax.ShapeDtypeStruct((N, 3 * C), f32),
                       jax.ShapeDtypeStruct((N, 2 * C), f32)],
        )(nf, Wq[l], _row(bq[l]), Wk[l], _row(bk[l]), Wv[l], _row(bv[l]))

        gdst, gsrc = _make_sc_gather()(tdst, tsrc, dst_g, src_g)

        msg = pl.pallas_call(
            _edge_body,
            grid=(EP // BE,),
            in_specs=[
                pl.BlockSpec((BE, 3 * C), lambda i: (i, 0)),
                pl.BlockSpec((BE, 2 * C), lambda i: (i, 0)),
                pl.BlockSpec((BE, NF), lambda i: (i, 0)),
                pl.BlockSpec((NF, C), lambda i: (0, 0)),
                pl.BlockSpec((3 * C, 3 * C), lambda i: (0, 0)),
                pl.BlockSpec((1, 3 * C), lambda i: (0, 0)),
                pl.BlockSpec((3 * C, C), lambda i: (0, 0)),
                pl.BlockSpec((1, C), lambda i: (0, 0)),
                pl.BlockSpec((1, 3 * C), lambda i: (0, 0)),
                pl.BlockSpec((1, 3 * C), lambda i: (0, 0)),
                pl.BlockSpec((1, C), lambda i: (0, 0)),
                pl.BlockSpec((1, C), lambda i: (0, 0)),
            ],
            out_specs=pl.BlockSpec((BE, C), lambda i: (i, 0)),
            out_shape=jax.ShapeDtypeStruct((EP, C), f32),
        )(gdst, gsrc, ef, We[l], Wm[l], _row(bm[l]), Wmsg[l], _row(bmsg[l]),
          _row(ln_g[l]), _row(ln_b[l]), _row(mln_g[l]), _row(mln_b[l]))

        partials = _make_sc_scatter()(msg, dst_s, zeros_blk)

        nf = pl.pallas_call(
            _update_body,
            out_shape=jax.ShapeDtypeStruct((N, NF), f32),
        )(partials, Wc[l], _row(bc[l]), _row(bn_g[l]), _row(bn_b[l]))

    out = pl.pallas_call(
        _pool_body,
        out_shape=jax.ShapeDtypeStruct((128, 1), f32),
    )(nf, batch.reshape(N, 1), W_fc, _row(b_fc), W_out, _row(b_out))

    return out[:G, 0]
